# baseline (speedup 1.0000x reference)
import functools

import jax
import jax.numpy as jnp
from jax import lax
from jax.experimental import pallas as pl
from jax.experimental.pallas import tpu as pltpu
from jax.experimental.pallas import tpu_sc as plsc

_A = 128
_R = 4
_K = _A * _R

_BS = 2048
_NB = _BS // 128

_NW = 32
_BB_PER_W = 4


def _tc_body(q_ref, w_ref, i_ref):
    qb = q_ref[:]
    q2 = qb.reshape(_BS, _K)
    s = [q2[:, r * _A:(r + 1) * _A] for r in range(_R)]

    wt = jnp.transpose(w_ref[:])
    wblk = jnp.concatenate([wt[:, _R * bb:_R * (bb + 1)]
                            for bb in range(_NB)], axis=0)

    prod = (s[0] * wblk[:, 0:1] + s[1] * wblk[:, 1:2]
            + s[2] * wblk[:, 2:3] + s[3] * wblk[:, 3:4])
    a_star = jnp.argmax(prod, axis=1).astype(jnp.int32)
    i_ref[:] = jnp.transpose(a_star[:, None]).reshape(_NB, _A)


def _sc_gather(qflat, astar):
    mesh = plsc.VectorSubcoreMesh(core_axis_name="c", subcore_axis_name="s")

    @functools.partial(
        pl.kernel,
        mesh=mesh,
        out_type=jax.ShapeDtypeStruct((16384 * _R,), jnp.float32),
        scratch_types=[
            pltpu.VMEM((_A,), jnp.int32),
            pltpu.VMEM((_K,), jnp.int32),
            pltpu.VMEM((_K,), jnp.float32),
            pltpu.SemaphoreType.DMA,
        ],
    )
    def gath(q_hbm, a_hbm, o_hbm, a_v, idx_v, rows_v, sem):
        wid = lax.axis_index("s") * 2 + lax.axis_index("c")
        lane = lax.broadcasted_iota(jnp.int32, (16,), 0)
        for j in range(_BB_PER_W):
            bb = wid * _BB_PER_W + j
            pltpu.sync_copy(a_hbm.at[bb], a_v)
            for r in range(_R):
                for v in range(_A // 16):
                    b_in = lane + 16 * v
                    a_chunk = a_v[pl.ds(16 * v, 16)]
                    idx = (bb * _A + b_in) * _K + r * _A + a_chunk
                    idx_v[pl.ds(r * _A + 16 * v, 16)] = idx
            pltpu.async_copy(q_hbm.at[idx_v], rows_v, sem).wait()
            pltpu.sync_copy(rows_v, o_hbm.at[pl.ds(bb * _K, _K)])

    return gath(qflat, astar)


@jax.jit
def kernel(q, w):
    bq = q.shape[0] // _A
    qt2 = q.reshape(bq, _A, _R).transpose(0, 2, 1).reshape(bq * _R, _A)
    wt2 = w.reshape(bq // _A, _A, _R).transpose(0, 2, 1).reshape(bq * _R // _A, _A)
    grid = (bq // _BS,)
    astar = pl.pallas_call(
        _tc_body,
        grid=grid,
        in_specs=[
            pl.BlockSpec((_BS * _R, _A), lambda i: (i, 0)),
            pl.BlockSpec((_R * _NB, _A), lambda i: (i, 0)),
        ],
        out_specs=pl.BlockSpec((_NB, _A), lambda i: (i, 0)),
        out_shape=jax.ShapeDtypeStruct((bq // _A, _A), jnp.int32),
        compiler_params=pltpu.CompilerParams(
            dimension_semantics=("arbitrary",),
        ),
    )(qt2, wt2)

    mo = _sc_gather(qt2.reshape(-1), astar)
    return mo.reshape(bq // _A, _R, _A).transpose(0, 2, 1).reshape(bq, _R)


# speedup vs baseline: 1.3177x; 1.3177x over previous
import jax
import jax.numpy as jnp
from jax.experimental import pallas as pl
from jax.experimental.pallas import tpu as pltpu

_A = 128
_R = 4
_K = _A * _R

_BS = 2048
_NB = _BS // 128


def _body(q_ref, w_ref, o_ref):
    qb = q_ref[:]
    q2 = qb.reshape(_BS, _K)
    s = [q2[:, r * _A:(r + 1) * _A] for r in range(_R)]

    wt = jnp.transpose(w_ref[:])
    wblk = jnp.concatenate([wt[:, _R * bb:_R * (bb + 1)]
                            for bb in range(_NB)], axis=0)

    prod = (s[0] * wblk[:, 0:1] + s[1] * wblk[:, 1:2]
            + s[2] * wblk[:, 2:3] + s[3] * wblk[:, 3:4])
    a_star = jnp.argmax(prod, axis=1).astype(jnp.int32)

    iota = jax.lax.broadcasted_iota(jnp.int32, (_BS, _A), 1)
    oh = iota == a_star[:, None]
    moq = jnp.concatenate(
        [jnp.sum(jnp.where(oh, s[r], 0.0), axis=1, keepdims=True)
         for r in range(_R)], axis=1)
    moqw = jnp.concatenate([moq[_A * bb:_A * (bb + 1), :]
                            for bb in range(_NB)], axis=1)
    o_ref[:] = jnp.transpose(moqw)


@jax.jit
def kernel(q, w):
    bq = q.shape[0] // _A
    qt2 = q.reshape(bq, _A, _R).transpose(0, 2, 1).reshape(bq * _R, _A)
    wt2 = w.reshape(bq // _A, _A, _R).transpose(0, 2, 1).reshape(bq * _R // _A, _A)
    grid = (bq // _BS,)
    ot = pl.pallas_call(
        _body,
        grid=grid,
        in_specs=[
            pl.BlockSpec((_BS * _R, _A), lambda i: (i, 0)),
            pl.BlockSpec((_R * _NB, _A), lambda i: (i, 0)),
        ],
        out_specs=pl.BlockSpec((_R * _NB, _A), lambda i: (i, 0)),
        out_shape=jax.ShapeDtypeStruct((bq * _R // _A, _A), jnp.float32),
        compiler_params=pltpu.CompilerParams(
            dimension_semantics=("parallel",),
        ),
    )(qt2, wt2)
    return ot.reshape(bq // _A, _R, _A).transpose(0, 2, 1).reshape(bq, _R)
